# baseline (speedup 1.0000x reference)
import functools

import jax
import jax.numpy as jnp
from jax import lax
from jax.experimental import pallas as pl
from jax.experimental.pallas import tpu as pltpu
from jax.experimental.pallas import tpu_sc as plsc

N = 8192
DIM = 64
H = 4
E = 16
M = 64
NBALLS = N // M
SCALE = DIM ** (-0.5)
ASCALE = E ** (-0.5)
KBOUND = 16.0
NEG = -1e30

NTILES = 32
PTS = N * H // NTILES
PARTS = NTILES // H
GRP = 8
PAD = PTS + 16


def _prep_body(x_ref, pos_ref, xeh_ref, kmean_ref, ktb_ref):
    xb = x_ref[...]
    pb = pos_ref[...]
    pmean = jnp.mean(pb.reshape(NBALLS, M, DIM), axis=1, keepdims=True)
    xe = xb + pb - jnp.broadcast_to(
        pmean, (NBALLS, M, DIM)).reshape(N, DIM)
    kmean_ref[...] = jnp.mean(xe.reshape(NBALLS, M, DIM), axis=1)
    for h in range(H):
        sl = xe[:, h * E:(h + 1) * E]
        xeh_ref[h, :, :] = sl
        sl3 = sl.reshape(N // 32, 32, E)
        a = jnp.transpose(sl3[:, 0:16, :].reshape(N // 2, E))
        bwd = jnp.transpose(sl3[:, 16:32, :].reshape(N // 2, E))
        abits = jax.lax.bitcast_convert_type(
            a.astype(jnp.bfloat16), jnp.uint16).astype(jnp.int32)
        bbits = jax.lax.bitcast_convert_type(
            bwd.astype(jnp.bfloat16), jnp.uint16).astype(jnp.int32)
        ktb_ref[h, :, :, :] = ((bbits << 16) | abits).reshape(E, NBALLS, 32)


def _route_body(kmean_ref, q_ref, idx_ref, ms_ref, *, bq):
    cols = jax.lax.broadcasted_iota(jnp.int32, (bq, NBALLS), 1)
    for h in range(H):
        q = q_ref[h]
        km = kmean_ref[h]
        logits = jax.lax.dot_general(
            q, km, (((1,), (1,)), ((), ())),
            preferred_element_type=jnp.float32) * SCALE
        m1 = jnp.max(logits, axis=1, keepdims=True)
        i1 = jnp.min(jnp.where(logits >= m1, cols, NBALLS),
                     axis=1, keepdims=True)
        l2 = jnp.where(cols == i1, NEG, logits)
        m2 = jnp.max(l2, axis=1, keepdims=True)
        i2 = jnp.min(jnp.where(l2 >= m2, cols, NBALLS),
                     axis=1, keepdims=True)
        idx_ref[h, :, :] = i1 * NBALLS + i2
        ms_ref[h, :, :] = (KBOUND * ASCALE) * jnp.sqrt(
            jnp.sum(q * q, axis=1, keepdims=True))


def _sc_attn_body(kt_hbm, q_hbm, idx_hbm, ms_hbm, out_hbm,
                  kt_v, q_v, idx_v, ms_v, out_v):
    wid = lax.axis_index("s") * 2 + lax.axis_index("c")
    head = wid // PARTS
    part = lax.rem(wid, PARTS)
    pltpu.sync_copy(kt_hbm.at[head], kt_v)
    pltpu.sync_copy(q_hbm.at[wid], q_v)
    pltpu.sync_copy(idx_hbm.at[wid], idx_v.at[pl.ds(0, PTS)])
    pltpu.sync_copy(ms_hbm.at[wid], ms_v.at[pl.ds(0, PTS)])

    lane16 = jax.lax.iota(jnp.int32, 16)

    def unpk(w):
        lo = jax.lax.bitcast_convert_type(w << 16, jnp.float32)
        hi = jax.lax.bitcast_convert_type(w & jnp.int32(-65536), jnp.float32)
        return lo, hi

    @plsc.parallel_loop(0, PTS // GRP)
    def _grp(g):
        base = pl.multiple_of(g * GRP, GRP)
        idx16 = idx_v[pl.ds(base, 16)]
        ms16 = ms_v[pl.ds(base, 16)]
        for j in range(GRP):
            pidx = base + j
            packed = idx16[j]
            b1 = packed // NBALLS
            b2 = lax.rem(packed, NBALLS)
            msafe = ms16[j]
            qv = q_v[pidx, :] * ASCALE
            qs = [qv[e] for e in range(E)]

            l_acc = jnp.zeros((16,), jnp.float32)
            accs = [jnp.zeros((16,), jnp.float32) for _ in range(E)]
            for b in (b1, b2):
                for w in range(2):
                    regs = [kt_v[e, b, pl.ds(w * 16, 16)] for e in range(E)]
                    s_lo = jnp.zeros((16,), jnp.float32)
                    s_hi = jnp.zeros((16,), jnp.float32)
                    for e in range(E):
                        lo, hi = unpk(regs[e])
                        s_lo += qs[e] * lo
                        s_hi += qs[e] * hi
                    p_lo = jnp.exp(s_lo - msafe)
                    p_hi = jnp.exp(s_hi - msafe)
                    l_acc = l_acc + p_lo + p_hi
                    for e in range(E):
                        lo, hi = unpk(regs[e])
                        accs[e] = accs[e] + p_lo * lo + p_hi * hi

            l = jnp.sum(l_acc)
            u = jnp.zeros((16,), jnp.float32)
            for e in range(E):
                u = u + jnp.where(lane16 == e, jnp.sum(accs[e]), 0.0)
            denom = jnp.zeros((16,), jnp.float32) + l
            out_v[pidx, :] = u / denom

    pltpu.sync_copy(out_v, out_hbm.at[pl.ds(part * PTS, PTS), head])


@jax.jit
def kernel(x, pos):
    xeh, kmean, ktb = pl.pallas_call(
        _prep_body,
        out_shape=[
            jax.ShapeDtypeStruct((H, N, E), jnp.float32),
            jax.ShapeDtypeStruct((NBALLS, DIM), jnp.float32),
            jax.ShapeDtypeStruct((H, E, NBALLS, M // 2), jnp.int32),
        ],
    )(x, pos)
    kmeanh = jnp.transpose(kmean.reshape(NBALLS, H, E), (1, 0, 2))

    bq = 512
    idx3, ms3 = pl.pallas_call(
        functools.partial(_route_body, bq=bq),
        grid=(N // bq,),
        in_specs=[
            pl.BlockSpec((H, NBALLS, E), lambda qi: (0, 0, 0)),
            pl.BlockSpec((H, bq, E), lambda qi: (0, qi, 0)),
        ],
        out_specs=[
            pl.BlockSpec((H, bq, 1), lambda qi: (0, qi, 0)),
            pl.BlockSpec((H, bq, 1), lambda qi: (0, qi, 0)),
        ],
        out_shape=[
            jax.ShapeDtypeStruct((H, N, 1), jnp.int32),
            jax.ShapeDtypeStruct((H, N, 1), jnp.float32),
        ],
    )(kmeanh, xeh)

    q32 = xeh.reshape(NTILES, PTS, E)
    idx32 = idx3.reshape(NTILES, PTS)
    ms32 = ms3.reshape(NTILES, PTS)

    mesh = plsc.VectorSubcoreMesh(core_axis_name="c", subcore_axis_name="s")
    sc_attn = functools.partial(
        pl.kernel,
        out_type=jax.ShapeDtypeStruct((N, H, E), jnp.float32),
        mesh=mesh,
        compiler_params=pltpu.CompilerParams(
            needs_layout_passes=False, use_tc_tiling_on_sc=False),
        scratch_types=[
            pltpu.VMEM((E, NBALLS, M // 2), jnp.int32),
            pltpu.VMEM((PTS, E), jnp.float32),
            pltpu.VMEM((PAD,), jnp.int32),
            pltpu.VMEM((PAD,), jnp.float32),
            pltpu.VMEM((PTS, E), jnp.float32),
        ],
    )(_sc_attn_body)
    return sc_attn(ktb, q32, idx32, ms32)


# speedup vs baseline: 1.0322x; 1.0322x over previous
import functools

import jax
import jax.numpy as jnp
from jax import lax
from jax.experimental import pallas as pl
from jax.experimental.pallas import tpu as pltpu
from jax.experimental.pallas import tpu_sc as plsc

N = 8192
DIM = 64
H = 4
E = 16
M = 64
NBALLS = N // M
SCALE = DIM ** (-0.5)
ASCALE = E ** (-0.5)
KBOUND = 16.0
NEG = -1e30

NTILES = 32
PTS = N * H // NTILES
PARTS = NTILES // H
GRP = 8
PAD = PTS + 16


def _prep_body(x_ref, pos_ref, xeh_ref, kmean_ref, ktb_ref):
    xb = x_ref[...]
    pb = pos_ref[...]
    pmean = jnp.mean(pb.reshape(NBALLS, M, DIM), axis=1, keepdims=True)
    xe = xb + pb - jnp.broadcast_to(
        pmean, (NBALLS, M, DIM)).reshape(N, DIM)
    kmean_ref[...] = jnp.mean(xe.reshape(NBALLS, M, DIM), axis=1)
    for h in range(H):
        sl = xe[:, h * E:(h + 1) * E]
        xeh_ref[h, :, :] = sl
        sl3 = sl.reshape(N // 32, 32, E)
        a = jnp.transpose(sl3[:, 0:16, :].reshape(N // 2, E))
        bwd = jnp.transpose(sl3[:, 16:32, :].reshape(N // 2, E))
        abits = jax.lax.bitcast_convert_type(
            a.astype(jnp.bfloat16), jnp.uint16).astype(jnp.int32)
        bbits = jax.lax.bitcast_convert_type(
            bwd.astype(jnp.bfloat16), jnp.uint16).astype(jnp.int32)
        ktb_ref[h, :, :, :] = ((bbits << 16) | abits).reshape(E, NBALLS, 32)


def _route_body(kmean_ref, q_ref, idx_ref, ms_ref, *, bq):
    cols = jax.lax.broadcasted_iota(jnp.int32, (bq, NBALLS), 1)
    for h in range(H):
        q = q_ref[h]
        km = kmean_ref[h]
        logits = jax.lax.dot_general(
            q, km, (((1,), (1,)), ((), ())),
            preferred_element_type=jnp.float32) * SCALE
        m1 = jnp.max(logits, axis=1, keepdims=True)
        i1 = jnp.min(jnp.where(logits >= m1, cols, NBALLS),
                     axis=1, keepdims=True)
        l2 = jnp.where(cols == i1, NEG, logits)
        m2 = jnp.max(l2, axis=1, keepdims=True)
        i2 = jnp.min(jnp.where(l2 >= m2, cols, NBALLS),
                     axis=1, keepdims=True)
        idx_ref[h, :, :] = i1 * NBALLS + i2
        ms_ref[h, :, :] = (KBOUND * ASCALE) * jnp.sqrt(
            jnp.sum(q * q, axis=1, keepdims=True))


def _sc_attn_body(kt_hbm, q_hbm, idx_hbm, ms_hbm, out_hbm,
                  kt_v, q_v, idx_v, ms_v, out_v):
    wid = lax.axis_index("s") * 2 + lax.axis_index("c")
    head = wid // PARTS
    part = lax.rem(wid, PARTS)
    pltpu.sync_copy(kt_hbm.at[head], kt_v)
    pltpu.sync_copy(q_hbm.at[wid], q_v)
    pltpu.sync_copy(idx_hbm.at[wid], idx_v.at[pl.ds(0, PTS)])
    pltpu.sync_copy(ms_hbm.at[wid], ms_v.at[pl.ds(0, PTS)])

    lane16 = jax.lax.iota(jnp.int32, 16)

    def unpk(w):
        lo = jax.lax.bitcast_convert_type(w << 16, jnp.float32)
        hi = jax.lax.bitcast_convert_type(w & jnp.int32(-65536), jnp.float32)
        return lo, hi

    @plsc.parallel_loop(0, PTS // GRP)
    def _grp(g):
        base = pl.multiple_of(g * GRP, GRP)
        idx16 = idx_v[pl.ds(base, 16)]
        ms16 = ms_v[pl.ds(base, 16)]
        for j in range(GRP):
            pidx = base + j
            packed = idx16[j]
            b1 = packed // NBALLS
            b2 = lax.rem(packed, NBALLS)
            msafe = ms16[j]
            qv = q_v[pidx, :] * ASCALE
            qs = [qv[e] for e in range(E)]

            l_acc = jnp.zeros((16,), jnp.float32)
            accs = [jnp.zeros((16,), jnp.float32) for _ in range(E)]
            for b in (b1, b2):
                s = [jnp.zeros((16,), jnp.float32) for _ in range(4)]
                for e in range(E):
                    a0, a1 = unpk(kt_v[e, b, pl.ds(0, 16)])
                    a2, a3 = unpk(kt_v[e, b, pl.ds(16, 16)])
                    s[0] += qs[e] * a0
                    s[1] += qs[e] * a1
                    s[2] += qs[e] * a2
                    s[3] += qs[e] * a3
                p = [jnp.exp(sc - msafe) for sc in s]
                l_acc = l_acc + p[0] + p[1] + p[2] + p[3]
                for e in range(E):
                    a0, a1 = unpk(kt_v[e, b, pl.ds(0, 16)])
                    a2, a3 = unpk(kt_v[e, b, pl.ds(16, 16)])
                    accs[e] = (accs[e] + p[0] * a0 + p[1] * a1
                               + p[2] * a2 + p[3] * a3)

            l = jnp.sum(l_acc)
            u = jnp.zeros((16,), jnp.float32)
            for e in range(E):
                u = u + jnp.where(lane16 == e, jnp.sum(accs[e]), 0.0)
            denom = jnp.zeros((16,), jnp.float32) + l
            out_v[pidx, :] = u / denom

    pltpu.sync_copy(out_v, out_hbm.at[pl.ds(part * PTS, PTS), head])


@jax.jit
def kernel(x, pos):
    xeh, kmean, ktb = pl.pallas_call(
        _prep_body,
        out_shape=[
            jax.ShapeDtypeStruct((H, N, E), jnp.float32),
            jax.ShapeDtypeStruct((NBALLS, DIM), jnp.float32),
            jax.ShapeDtypeStruct((H, E, NBALLS, M // 2), jnp.int32),
        ],
    )(x, pos)
    kmeanh = jnp.transpose(kmean.reshape(NBALLS, H, E), (1, 0, 2))

    bq = 512
    idx3, ms3 = pl.pallas_call(
        functools.partial(_route_body, bq=bq),
        grid=(N // bq,),
        in_specs=[
            pl.BlockSpec((H, NBALLS, E), lambda qi: (0, 0, 0)),
            pl.BlockSpec((H, bq, E), lambda qi: (0, qi, 0)),
        ],
        out_specs=[
            pl.BlockSpec((H, bq, 1), lambda qi: (0, qi, 0)),
            pl.BlockSpec((H, bq, 1), lambda qi: (0, qi, 0)),
        ],
        out_shape=[
            jax.ShapeDtypeStruct((H, N, 1), jnp.int32),
            jax.ShapeDtypeStruct((H, N, 1), jnp.float32),
        ],
    )(kmeanh, xeh)

    q32 = xeh.reshape(NTILES, PTS, E)
    idx32 = idx3.reshape(NTILES, PTS)
    ms32 = ms3.reshape(NTILES, PTS)

    mesh = plsc.VectorSubcoreMesh(core_axis_name="c", subcore_axis_name="s")
    sc_attn = functools.partial(
        pl.kernel,
        out_type=jax.ShapeDtypeStruct((N, H, E), jnp.float32),
        mesh=mesh,
        compiler_params=pltpu.CompilerParams(
            needs_layout_passes=False, use_tc_tiling_on_sc=False),
        scratch_types=[
            pltpu.VMEM((E, NBALLS, M // 2), jnp.int32),
            pltpu.VMEM((PTS, E), jnp.float32),
            pltpu.VMEM((PAD,), jnp.int32),
            pltpu.VMEM((PAD,), jnp.float32),
            pltpu.VMEM((PTS, E), jnp.float32),
        ],
    )(_sc_attn_body)
    return sc_attn(ktb, q32, idx32, ms32)
